# baseline (speedup 1.0000x reference)
import functools

import jax
import jax.numpy as jnp
from jax import lax
from jax.experimental import pallas as pl
from jax.experimental.pallas import tpu as pltpu
from jax.experimental.pallas import tpu_sc as plsc

_NW = 32


def kernel(input_ids, word_embeddings, position_embeddings):
    B, S = input_ids.shape
    V, D = word_embeddings.shape
    L = 16
    SP = ((S + L - 1) // L) * L + L
    bpw = B // _NW
    word2 = jnp.pad(word_embeddings, ((0, 0), (0, D)))
    pos_flat = position_embeddings.reshape(-1)
    mesh = plsc.VectorSubcoreMesh(core_axis_name="c", subcore_axis_name="s")

    @functools.partial(
        pl.kernel,
        mesh=mesh,
        out_type=jax.ShapeDtypeStruct((B, S, D), jnp.float32),
        scratch_types=[
            [pltpu.VMEM((SP,), jnp.int32) for _ in range(2)],
            [pltpu.VMEM((S, 2 * D), jnp.float32) for _ in range(2)],
            [pltpu.VMEM((S, D), jnp.float32) for _ in range(2)],
            pltpu.VMEM((S * D,), jnp.float32),
            [pltpu.SemaphoreType.DMA for _ in range(2)],
            [pltpu.SemaphoreType.DMA for _ in range(2)],
        ],
    )
    def body(ids_hbm, word_hbm, pos_hbm, out_hbm,
             idxs, rows, obufs, pos_v, gsems, ssems):
        wid = lax.axis_index("s") * 2 + lax.axis_index("c")
        base = wid * bpw
        pltpu.sync_copy(pos_hbm.at[pl.ds(0, S * D)], pos_v)

        def prep_chunk(ci, g):
            pltpu.sync_copy(
                ids_hbm.at[pl.ds((base + ci) * S, S)], idxs[g].at[pl.ds(0, S)]
            )
            h1 = (S // 2 + 7) // 8 * 8
            pltpu.async_copy(
                word_hbm.at[idxs[g].at[pl.ds(0, h1)]],
                rows[g].at[pl.ds(0, h1)], gsems[g],
            )
            pltpu.async_copy(
                word_hbm.at[idxs[g].at[pl.ds(h1, S - h1)]],
                rows[g].at[pl.ds(h1, S - h1)], gsems[g],
            )

        def wait_gather(g):
            pltpu.make_async_copy(
                word_hbm.at[idxs[g].at[pl.ds(0, S)]], rows[g], gsems[g]
            ).wait()

        def issue_scatter(ci, ob):
            pltpu.async_copy(obufs[ob], out_hbm.at[base + ci], ssems[ob])

        def wait_scatter(ob):
            pltpu.make_async_copy(
                obufs[ob], out_hbm.at[base], ssems[ob]
            ).wait()

        def pack_add(g, ob):
            buf = rows[g]
            obuf = obufs[ob]

            @plsc.parallel_loop(0, S, 1, unroll=8)
            def _(r):
                for k in range(D // L):
                    sl = pl.ds(k * L, L)
                    obuf[r, sl] = buf[r, sl] + pos_v[pl.ds(r * D + k * L, L)]

        prep_chunk(0, 0)

        @pl.loop(0, bpw, step=2)
        def _(c0):
            for j in range(2):
                ci = c0 + j

                @pl.when(ci + 1 < bpw)
                def _():
                    prep_chunk(ci + 1, 1 - j)

                wait_gather(j)

                @pl.when(ci >= 2)
                def _():
                    wait_scatter(j)

                pack_add(j, j)
                issue_scatter(ci, j)

        for ob in range(2):
            wait_scatter(ob)

    return body(input_ids.reshape(-1), word2, pos_flat)


# speedup vs baseline: 1.0016x; 1.0016x over previous
import functools

import jax
import jax.numpy as jnp
from jax import lax
from jax.experimental import pallas as pl
from jax.experimental.pallas import tpu as pltpu
from jax.experimental.pallas import tpu_sc as plsc

_NW = 32


def kernel(input_ids, word_embeddings, position_embeddings):
    B, S = input_ids.shape
    V, D = word_embeddings.shape
    L = 16
    SP = ((S + L - 1) // L) * L + L
    bpw = B // _NW
    word2 = jnp.pad(word_embeddings.T, ((0, D), (0, 0))).T
    pos_flat = position_embeddings.reshape(-1)
    mesh = plsc.VectorSubcoreMesh(core_axis_name="c", subcore_axis_name="s")

    @functools.partial(
        pl.kernel,
        mesh=mesh,
        out_type=jax.ShapeDtypeStruct((B, S, D), jnp.float32),
        scratch_types=[
            [pltpu.VMEM((SP,), jnp.int32) for _ in range(2)],
            [pltpu.VMEM((S, 2 * D), jnp.float32) for _ in range(2)],
            [pltpu.VMEM((S, D), jnp.float32) for _ in range(2)],
            pltpu.VMEM((S * D,), jnp.float32),
            [pltpu.SemaphoreType.DMA for _ in range(2)],
            [pltpu.SemaphoreType.DMA for _ in range(2)],
        ],
    )
    def body(ids_hbm, word_hbm, pos_hbm, out_hbm,
             idxs, rows, obufs, pos_v, gsems, ssems):
        wid = lax.axis_index("s") * 2 + lax.axis_index("c")
        base = wid * bpw
        pltpu.sync_copy(pos_hbm.at[pl.ds(0, S * D)], pos_v)

        def prep_chunk(ci, g):
            pltpu.sync_copy(
                ids_hbm.at[pl.ds((base + ci) * S, S)], idxs[g].at[pl.ds(0, S)]
            )
            h1 = (S // 2 + 7) // 8 * 8
            pltpu.async_copy(
                word_hbm.at[idxs[g].at[pl.ds(0, h1)]],
                rows[g].at[pl.ds(0, h1)], gsems[g],
            )
            pltpu.async_copy(
                word_hbm.at[idxs[g].at[pl.ds(h1, S - h1)]],
                rows[g].at[pl.ds(h1, S - h1)], gsems[g],
            )

        def wait_gather(g):
            pltpu.make_async_copy(
                word_hbm.at[idxs[g].at[pl.ds(0, S)]], rows[g], gsems[g]
            ).wait()

        def issue_scatter(ci, ob):
            pltpu.async_copy(obufs[ob], out_hbm.at[base + ci], ssems[ob])

        def wait_scatter(ob):
            pltpu.make_async_copy(
                obufs[ob], out_hbm.at[base], ssems[ob]
            ).wait()

        def pack_add(g, ob):
            buf = rows[g]
            obuf = obufs[ob]

            @plsc.parallel_loop(0, S, 1, unroll=8)
            def _(r):
                for k in range(D // L):
                    sl = pl.ds(k * L, L)
                    obuf[r, sl] = buf[r, sl] + pos_v[pl.ds(r * D + k * L, L)]

        prep_chunk(0, 0)

        @pl.loop(0, bpw, step=2)
        def _(c0):
            for j in range(2):
                ci = c0 + j

                @pl.when(ci + 1 < bpw)
                def _():
                    prep_chunk(ci + 1, 1 - j)

                wait_gather(j)

                @pl.when(ci >= 2)
                def _():
                    wait_scatter(j)

                pack_add(j, j)
                issue_scatter(ci, j)

        for ob in range(2):
            wait_scatter(ob)

    return body(input_ids.reshape(-1), word2, pos_flat)


# speedup vs baseline: 1.0060x; 1.0044x over previous
import functools

import jax
import jax.numpy as jnp
from jax import lax
from jax.experimental import pallas as pl
from jax.experimental.pallas import tpu as pltpu
from jax.experimental.pallas import tpu_sc as plsc

_NW = 32


def kernel(input_ids, word_embeddings, position_embeddings):
    B, S = input_ids.shape
    V, D = word_embeddings.shape
    L = 16
    SP = ((S + L - 1) // L) * L + L
    bpw = B // _NW
    word2 = jnp.pad(word_embeddings, ((0, 0), (0, D)))
    pos_flat = position_embeddings.reshape(-1)
    mesh = plsc.VectorSubcoreMesh(core_axis_name="c", subcore_axis_name="s")

    @functools.partial(
        pl.kernel,
        mesh=mesh,
        out_type=jax.ShapeDtypeStruct((B, S, D), jnp.float32),
        scratch_types=[
            [pltpu.VMEM((SP,), jnp.int32) for _ in range(4)],
            [pltpu.VMEM((S, 2 * D), jnp.float32) for _ in range(2)],
            [pltpu.VMEM((S, D), jnp.float32) for _ in range(2)],
            pltpu.VMEM((S * D,), jnp.float32),
            [pltpu.SemaphoreType.DMA for _ in range(4)],
            [pltpu.SemaphoreType.DMA for _ in range(2)],
            [pltpu.SemaphoreType.DMA for _ in range(2)],
        ],
    )
    def body(ids_hbm, word_hbm, pos_hbm, out_hbm,
             idxs, rows, obufs, pos_v, isems, gsems, ssems):
        wid = lax.axis_index("s") * 2 + lax.axis_index("c")
        base = wid * bpw
        pltpu.sync_copy(pos_hbm.at[pl.ds(0, S * D)], pos_v)

        def issue_ids(ci, ib):
            pltpu.async_copy(
                ids_hbm.at[pl.ds((base + ci) * S, S)],
                idxs[ib].at[pl.ds(0, S)], isems[ib],
            )

        def wait_ids(ib):
            pltpu.make_async_copy(
                ids_hbm.at[pl.ds(base * S, S)],
                idxs[ib].at[pl.ds(0, S)], isems[ib],
            ).wait()

        def issue_gather(g, ib):
            h1 = (S // 2 + 7) // 8 * 8
            pltpu.async_copy(
                word_hbm.at[idxs[ib].at[pl.ds(0, h1)]],
                rows[g].at[pl.ds(0, h1)], gsems[g],
            )
            pltpu.async_copy(
                word_hbm.at[idxs[ib].at[pl.ds(h1, S - h1)]],
                rows[g].at[pl.ds(h1, S - h1)], gsems[g],
            )

        def wait_gather(g):
            pltpu.make_async_copy(
                word_hbm.at[idxs[0].at[pl.ds(0, S)]], rows[g], gsems[g]
            ).wait()

        def issue_scatter(ci, ob):
            pltpu.async_copy(obufs[ob], out_hbm.at[base + ci], ssems[ob])

        def wait_scatter(ob):
            pltpu.make_async_copy(
                obufs[ob], out_hbm.at[base], ssems[ob]
            ).wait()

        def pack_add(g, ob):
            buf = rows[g]
            obuf = obufs[ob]

            @plsc.parallel_loop(0, S, 1, unroll=8)
            def _(r):
                for k in range(D // L):
                    sl = pl.ds(k * L, L)
                    obuf[r, sl] = buf[r, sl] + pos_v[pl.ds(r * D + k * L, L)]

        issue_ids(0, 0)
        issue_ids(1, 1)
        wait_ids(0)
        issue_gather(0, 0)
        issue_ids(2, 2)

        @pl.loop(0, bpw, step=4)
        def _(c0):
            for b in range(4):
                ci = c0 + b
                j = b % 2

                @pl.when(ci + 1 < bpw)
                def _():
                    wait_ids((b + 1) % 4)
                    issue_gather(1 - j, (b + 1) % 4)

                wait_gather(j)

                @pl.when(ci + 3 < bpw)
                def _():
                    issue_ids(ci + 3, (b + 3) % 4)

                @pl.when(ci >= 2)
                def _():
                    wait_scatter(j)

                pack_add(j, j)
                issue_scatter(ci, j)

        for ob in range(2):
            wait_scatter(ob)

    return body(input_ids.reshape(-1), word2, pos_flat)
